# baseline (speedup 1.0000x reference)
import jax
import jax.numpy as jnp
from jax.experimental import pallas as pl
from jax.experimental.pallas import tpu as pltpu
from jax.experimental.pallas import tpu_sc as plsc

_B, _L = 1024, 32
_V, _D, _LV = 8192, 64, 1000
_BL = _B * _L

_DP = 128
_TOK_WIN = 128
_CTX_WIN = 128

_ROWS = 512
_NBATCH = _ROWS // _L

_NCHUNK = 4
_CROWS = _BL // _NCHUNK
_CSTEPS = _CROWS // _ROWS

_SC_MESH = plsc.VectorSubcoreMesh(core_axis_name="c", subcore_axis_name="s")


def _gather_pipeline(table_hbm, idx_hbm, out_hbm, n_rows, win):
    def body(i_vmem, o_vmem):
        pltpu.sync_copy(table_hbm.at[i_vmem.at[0]], o_vmem)

    pltpu.emit_pipeline(
        body,
        grid=(n_rows // win,),
        in_specs=[pl.BlockSpec((1, win), index_map=lambda i: (0, i))],
        out_specs=[pl.BlockSpec((win, _DP), index_map=lambda i: (i, 0))],
        core_axis_name=("c", "s"),
        dimension_semantics=(pltpu.PARALLEL,),
    )(idx_hbm, out_hbm)


def _sc_gather_tok(tok_pad, ids_chunk):
    n = ids_chunk.shape[1]

    def body(tok_hbm, ids_hbm, otok_hbm):
        _gather_pipeline(tok_hbm, ids_hbm, otok_hbm, n, _TOK_WIN)

    f = pl.kernel(
        body,
        out_type=jax.ShapeDtypeStruct((n, _DP), jnp.float32),
        mesh=_SC_MESH,
    )
    return f(tok_pad, ids_chunk)


def _sc_gather_lab(lab_pad, ctx_flat):
    def body(lab_hbm, ctx_hbm, olab_hbm):
        _gather_pipeline(lab_hbm, ctx_hbm, olab_hbm, _B, _CTX_WIN)

    f = pl.kernel(
        body,
        out_type=jax.ShapeDtypeStruct((_B, _DP), jnp.float32),
        mesh=_SC_MESH,
    )
    return f(lab_pad, ctx_flat)


def _proj_body(tok_ref, lab_ref, wt_ref, b_ref, o_ref):
    tok = tok_ref[...][:, :_D].reshape(_NBATCH, _L, _D)
    lab = lab_ref[...][:, :_D]
    x = (tok + lab[:, None, :]).reshape(_ROWS, _D).astype(jnp.bfloat16)
    acc = jnp.dot(x, wt_ref[...], preferred_element_type=jnp.float32)
    o_ref[...] = acc + b_ref[...]


def _proj_body_aliased(tok_ref, lab_ref, wt_ref, b_ref, _prev_ref, o_ref):
    _proj_body(tok_ref, lab_ref, wt_ref, b_ref, o_ref)


def _project_chunk(chunk, tok_x, lab_x, wt, b2d, prev_out):
    row0 = chunk * _CSTEPS
    in_specs = [
        pl.BlockSpec((_ROWS, _DP), lambda i: (i, 0)),
        pl.BlockSpec((_NBATCH, _DP), lambda i, c=chunk: (c * _CSTEPS + i, 0)),
        pl.BlockSpec((_D, _V), lambda i: (0, 0)),
        pl.BlockSpec((1, _V), lambda i: (0, 0)),
    ]
    args = [tok_x, lab_x, wt, b2d]
    kwargs = {}
    body = _proj_body
    if prev_out is not None:
        in_specs.append(pl.BlockSpec(memory_space=pl.ANY))
        args.append(prev_out)
        kwargs["input_output_aliases"] = {4: 0}
        body = _proj_body_aliased
    return pl.pallas_call(
        body,
        grid=(_CSTEPS,),
        in_specs=in_specs,
        out_specs=pl.BlockSpec((_ROWS, _V), lambda i, r0=row0: (r0 + i, 0)),
        out_shape=jax.ShapeDtypeStruct((_BL, _V), jnp.float32),
        compiler_params=pltpu.CompilerParams(
            dimension_semantics=("arbitrary",),
        ),
        **kwargs,
    )(*args)


def kernel(input_ids, context, token_embed, label_embed, W, b):
    wt = W.astype(jnp.bfloat16).T
    ids_flat = input_ids.reshape(1, _BL).astype(jnp.int32)
    ctx_flat = context.reshape(1, _B).astype(jnp.int32)
    tok_pad = jnp.pad(token_embed, ((0, 0), (0, _DP - _D)))
    lab_pad = jnp.pad(label_embed, ((0, 0), (0, _DP - _D)))
    b2d = b.reshape(1, _V)

    lab_x = _sc_gather_lab(lab_pad, ctx_flat)
    out = None
    for k in range(_NCHUNK):
        ids_k = jax.lax.slice(ids_flat, (0, k * _CROWS), (1, (k + 1) * _CROWS))
        tok_x = _sc_gather_tok(tok_pad, ids_k)
        out = _project_chunk(k, tok_x, lab_x, wt, b2d, out)
    return out.reshape(_B, _L, _V)


# speedup vs baseline: 1.0314x; 1.0314x over previous
import jax
import jax.numpy as jnp
from jax.experimental import pallas as pl
from jax.experimental.pallas import tpu as pltpu
from jax.experimental.pallas import tpu_sc as plsc

_B, _L = 1024, 32
_V, _D, _LV = 8192, 64, 1000
_BL = _B * _L

_DP = 128
_TOK_WIN = 128
_CTX_WIN = 128

_ROWS = 512
_NBATCH = _ROWS // _L

_NCHUNK = 2
_CROWS = _BL // _NCHUNK
_CSTEPS = _CROWS // _ROWS

_SC_MESH = plsc.VectorSubcoreMesh(core_axis_name="c", subcore_axis_name="s")


def _gather_pipeline(table_hbm, idx_hbm, out_hbm, n_rows, win):
    def body(i_vmem, o_vmem):
        pltpu.sync_copy(table_hbm.at[i_vmem.at[0]], o_vmem)

    pltpu.emit_pipeline(
        body,
        grid=(n_rows // win,),
        in_specs=[pl.BlockSpec((1, win), index_map=lambda i: (0, i))],
        out_specs=[pl.BlockSpec((win, _DP), index_map=lambda i: (i, 0))],
        core_axis_name=("c", "s"),
        dimension_semantics=(pltpu.PARALLEL,),
    )(idx_hbm, out_hbm)


def _sc_gather_tok(tok_pad, ids_chunk):
    n = ids_chunk.shape[1]

    def body(tok_hbm, ids_hbm, otok_hbm):
        _gather_pipeline(tok_hbm, ids_hbm, otok_hbm, n, _TOK_WIN)

    f = pl.kernel(
        body,
        out_type=jax.ShapeDtypeStruct((n, _DP), jnp.float32),
        mesh=_SC_MESH,
    )
    return f(tok_pad, ids_chunk)


def _sc_gather_lab(lab_pad, ctx_flat):
    def body(lab_hbm, ctx_hbm, olab_hbm):
        _gather_pipeline(lab_hbm, ctx_hbm, olab_hbm, _B, _CTX_WIN)

    f = pl.kernel(
        body,
        out_type=jax.ShapeDtypeStruct((_B, _DP), jnp.float32),
        mesh=_SC_MESH,
    )
    return f(lab_pad, ctx_flat)


def _proj_body(tok_ref, lab_ref, wt_ref, b_ref, o_ref):
    tok = tok_ref[...][:, :_D].reshape(_NBATCH, _L, _D)
    lab = lab_ref[...][:, :_D]
    x = (tok + lab[:, None, :]).reshape(_ROWS, _D).astype(jnp.bfloat16)
    acc = jnp.dot(x, wt_ref[...], preferred_element_type=jnp.float32)
    o_ref[...] = acc + b_ref[...]


def _proj_body_aliased(tok_ref, lab_ref, wt_ref, b_ref, _prev_ref, o_ref):
    _proj_body(tok_ref, lab_ref, wt_ref, b_ref, o_ref)


def _project_chunk(chunk, tok_x, lab_x, wt, b2d, prev_out):
    row0 = chunk * _CSTEPS
    in_specs = [
        pl.BlockSpec((_ROWS, _DP), lambda i: (i, 0)),
        pl.BlockSpec((_NBATCH, _DP), lambda i, c=chunk: (c * _CSTEPS + i, 0)),
        pl.BlockSpec((_D, _V), lambda i: (0, 0)),
        pl.BlockSpec((1, _V), lambda i: (0, 0)),
    ]
    args = [tok_x, lab_x, wt, b2d]
    kwargs = {}
    body = _proj_body
    if prev_out is not None:
        in_specs.append(pl.BlockSpec(memory_space=pl.ANY))
        args.append(prev_out)
        kwargs["input_output_aliases"] = {4: 0}
        body = _proj_body_aliased
    return pl.pallas_call(
        body,
        grid=(_CSTEPS,),
        in_specs=in_specs,
        out_specs=pl.BlockSpec((_ROWS, _V), lambda i, r0=row0: (r0 + i, 0)),
        out_shape=jax.ShapeDtypeStruct((_BL, _V), jnp.float32),
        compiler_params=pltpu.CompilerParams(
            dimension_semantics=("arbitrary",),
        ),
        **kwargs,
    )(*args)


def kernel(input_ids, context, token_embed, label_embed, W, b):
    wt = W.astype(jnp.bfloat16).T
    ids_flat = input_ids.reshape(1, _BL).astype(jnp.int32)
    ctx_flat = context.reshape(1, _B).astype(jnp.int32)
    tok_pad = jnp.pad(token_embed, ((0, 0), (0, _DP - _D)))
    lab_pad = jnp.pad(label_embed, ((0, 0), (0, _DP - _D)))
    b2d = b.reshape(1, _V)

    lab_x = _sc_gather_lab(lab_pad, ctx_flat)
    out = None
    for k in range(_NCHUNK):
        ids_k = jax.lax.slice(ids_flat, (0, k * _CROWS), (1, (k + 1) * _CROWS))
        tok_x = _sc_gather_tok(tok_pad, ids_k)
        out = _project_chunk(k, tok_x, lab_x, wt, b2d, out)
    return out.reshape(_B, _L, _V)


# speedup vs baseline: 1.0327x; 1.0012x over previous
import jax
import jax.numpy as jnp
from jax.experimental import pallas as pl
from jax.experimental.pallas import tpu as pltpu
from jax.experimental.pallas import tpu_sc as plsc

_B, _L = 1024, 32
_V, _D, _LV = 8192, 64, 1000
_BL = _B * _L

_DP = 128
_TOK_WIN = 128
_CTX_WIN = 128

_ROWS = 512
_NBATCH = _ROWS // _L

_SC_MESH = plsc.VectorSubcoreMesh(core_axis_name="c", subcore_axis_name="s")


def _gather_pipeline(table_hbm, idx_hbm, out_hbm, n_rows, win):
    def body(i_vmem, o_vmem):
        pltpu.sync_copy(table_hbm.at[i_vmem.at[0]], o_vmem)

    pltpu.emit_pipeline(
        body,
        grid=(n_rows // win,),
        in_specs=[pl.BlockSpec((1, win), index_map=lambda i: (0, i))],
        out_specs=[pl.BlockSpec((win, _DP), index_map=lambda i: (i, 0))],
        core_axis_name=("c", "s"),
        dimension_semantics=(pltpu.PARALLEL,),
    )(idx_hbm, out_hbm)


def _sc_gather_body(tok_hbm, ids_hbm, lab_hbm, ctx_hbm, otok_hbm, olab_hbm):
    _gather_pipeline(tok_hbm, ids_hbm, otok_hbm, _BL, _TOK_WIN)
    _gather_pipeline(lab_hbm, ctx_hbm, olab_hbm, _B, _CTX_WIN)


def _sc_gather(tok_pad, ids_flat, lab_pad, ctx_flat):
    f = pl.kernel(
        _sc_gather_body,
        out_type=(
            jax.ShapeDtypeStruct((_BL, _DP), jnp.float32),
            jax.ShapeDtypeStruct((_B, _DP), jnp.float32),
        ),
        mesh=_SC_MESH,
    )
    return f(tok_pad, ids_flat, lab_pad, ctx_flat)


def _proj_body(tok_ref, lab_ref, wt_ref, b_ref, o_ref):
    tok = tok_ref[...][:, :_D].reshape(_NBATCH, _L, _D)
    lab = lab_ref[...][:, :_D]
    x = (tok + lab[:, None, :]).reshape(_ROWS, _D).astype(jnp.bfloat16)
    acc = jnp.dot(x, wt_ref[...], preferred_element_type=jnp.float32)
    o_ref[...] = acc + b_ref[...]


def _project(tok_x, lab_x, wt, b2d):
    return pl.pallas_call(
        _proj_body,
        grid=(_BL // _ROWS,),
        in_specs=[
            pl.BlockSpec((_ROWS, _DP), lambda i: (i, 0)),
            pl.BlockSpec((_NBATCH, _DP), lambda i: (i, 0)),
            pl.BlockSpec((_D, _V), lambda i: (0, 0)),
            pl.BlockSpec((1, _V), lambda i: (0, 0)),
        ],
        out_specs=pl.BlockSpec((_ROWS, _V), lambda i: (i, 0)),
        out_shape=jax.ShapeDtypeStruct((_BL, _V), jnp.float32),
        compiler_params=pltpu.CompilerParams(
            dimension_semantics=("arbitrary",),
        ),
    )(tok_x, lab_x, wt, b2d)


def kernel(input_ids, context, token_embed, label_embed, W, b):
    wt = W.astype(jnp.bfloat16).T
    ids_flat = input_ids.reshape(1, _BL).astype(jnp.int32)
    ctx_flat = context.reshape(1, _B).astype(jnp.int32)
    tok_pad = jnp.pad(token_embed, ((0, 0), (0, _DP - _D)))
    lab_pad = jnp.pad(label_embed, ((0, 0), (0, _DP - _D)))
    tok_x, lab_x = _sc_gather(tok_pad, ids_flat, lab_pad, ctx_flat)
    logits = _project(tok_x, lab_x, wt, b.reshape(1, _V))
    return logits.reshape(_B, _L, _V)


# speedup vs baseline: 1.0392x; 1.0063x over previous
import jax
import jax.numpy as jnp
from jax.experimental import pallas as pl
from jax.experimental.pallas import tpu as pltpu
from jax.experimental.pallas import tpu_sc as plsc

_B, _L = 1024, 32
_V, _D, _LV = 8192, 64, 1000
_BL = _B * _L

_DP = 128
_TOK_WIN = 256
_CTX_WIN = 128

_ROWS = 512
_NBATCH = _ROWS // _L

_SC_MESH = plsc.VectorSubcoreMesh(core_axis_name="c", subcore_axis_name="s")


def _gather_pipeline(table_hbm, idx_hbm, out_hbm, n_rows, win):
    def body(i_vmem, o_vmem):
        pltpu.sync_copy(table_hbm.at[i_vmem.at[0]], o_vmem)

    pltpu.emit_pipeline(
        body,
        grid=(n_rows // win,),
        in_specs=[pl.BlockSpec((1, win), index_map=lambda i: (0, i))],
        out_specs=[pl.BlockSpec((win, _DP), index_map=lambda i: (i, 0))],
        core_axis_name=("c", "s"),
        dimension_semantics=(pltpu.PARALLEL,),
    )(idx_hbm, out_hbm)


def _sc_gather_body(tok_hbm, ids_hbm, lab_hbm, ctx_hbm, otok_hbm, olab_hbm):
    _gather_pipeline(tok_hbm, ids_hbm, otok_hbm, _BL, _TOK_WIN)
    _gather_pipeline(lab_hbm, ctx_hbm, olab_hbm, _B, _CTX_WIN)


def _sc_gather(tok_pad, ids_flat, lab_pad, ctx_flat):
    f = pl.kernel(
        _sc_gather_body,
        out_type=(
            jax.ShapeDtypeStruct((_BL, _DP), jnp.float32),
            jax.ShapeDtypeStruct((_B, _DP), jnp.float32),
        ),
        mesh=_SC_MESH,
    )
    return f(tok_pad, ids_flat, lab_pad, ctx_flat)


def _proj_body(tok_ref, lab_ref, wt_ref, b_ref, o_ref):
    tok = tok_ref[...][:, :_D].reshape(_NBATCH, _L, _D)
    lab = lab_ref[...][:, :_D]
    x = (tok + lab[:, None, :]).reshape(_ROWS, _D).astype(jnp.bfloat16)
    acc = jnp.dot(x, wt_ref[...], preferred_element_type=jnp.float32)
    o_ref[...] = acc + b_ref[...]


def _project(tok_x, lab_x, wt, b2d):
    return pl.pallas_call(
        _proj_body,
        grid=(_BL // _ROWS,),
        in_specs=[
            pl.BlockSpec((_ROWS, _DP), lambda i: (i, 0)),
            pl.BlockSpec((_NBATCH, _DP), lambda i: (i, 0)),
            pl.BlockSpec((_D, _V), lambda i: (0, 0)),
            pl.BlockSpec((1, _V), lambda i: (0, 0)),
        ],
        out_specs=pl.BlockSpec((_ROWS, _V), lambda i: (i, 0)),
        out_shape=jax.ShapeDtypeStruct((_BL, _V), jnp.float32),
        compiler_params=pltpu.CompilerParams(
            dimension_semantics=("arbitrary",),
        ),
    )(tok_x, lab_x, wt, b2d)


def kernel(input_ids, context, token_embed, label_embed, W, b):
    wt = W.astype(jnp.bfloat16).T
    ids_flat = input_ids.reshape(1, _BL).astype(jnp.int32)
    ctx_flat = context.reshape(1, _B).astype(jnp.int32)
    tok_pad = jnp.pad(token_embed, ((0, 0), (0, _DP - _D)))
    lab_pad = jnp.pad(label_embed, ((0, 0), (0, _DP - _D)))
    tok_x, lab_x = _sc_gather(tok_pad, ids_flat, lab_pad, ctx_flat)
    logits = _project(tok_x, lab_x, wt, b.reshape(1, _V))
    return logits.reshape(_B, _L, _V)
